# baseline (speedup 1.0000x reference)
import jax
import jax.numpy as jnp
from jax import lax
from jax.experimental import pallas as pl
from jax.experimental.pallas import tpu as pltpu
from jax.experimental.pallas import tpu_sc as plsc

D_IN = 1024
D_LEARNT = 32768
BATCH = 128
N_DEAD = 4096

BLK = 2048
BLKZ = 1024

NC = 2
NS = 16
NW = NC * NS
J_PER_W = N_DEAD // NW
V_PER_W = D_LEARNT // NW
L = 16


def _z_body(u_ref, x_ref, o_ref):
    o_ref[...] = lax.dot_general(
        u_ref[...], x_ref[...], (((1,), (1,)), ((), ())),
        preferred_element_type=jnp.float32)


def _main_body(x_ref, w_ref, b_ref, m_ref, zf_ref, o_ref):
    acc = lax.dot_general(
        x_ref[...], w_ref[...], (((1,), (1,)), ((), ())),
        preferred_element_type=jnp.float32)
    zt = zf_ref[...].T
    pre = jnp.where(m_ref[...] > 0.5, zt, acc)
    o_ref[...] = jnp.maximum(pre + b_ref[...], 0.0)


def _sc_scatter_body(idx_hbm, zt_hbm, bias_hbm, ub_hbm,
                     zfull_hbm, mask_hbm, bnew_hbm,
                     idx_v, rows_v, all_idx_v, all_ub_v, mbuf_v, bbuf_v,
                     sem_r, sem_i, sem_u, sem_s, sem_b):
    wid = lax.axis_index("s") * NC + lax.axis_index("c")

    jbase = wid * J_PER_W
    rows_cp = pltpu.async_copy(
        zt_hbm.at[pl.ds(jbase, J_PER_W)], rows_v, sem_r)
    idx_all_cp = pltpu.async_copy(idx_hbm, all_idx_v, sem_i)
    ub_all_cp = pltpu.async_copy(ub_hbm, all_ub_v, sem_u)
    pltpu.sync_copy(idx_hbm.at[pl.ds(jbase, J_PER_W)], idx_v)

    vbase = wid * V_PER_W
    bias_cp = pltpu.async_copy(
        bias_hbm.at[pl.ds(vbase, V_PER_W)], bbuf_v, sem_b)

    rows_cp.wait()
    scatter_cp = pltpu.async_copy(rows_v, zfull_hbm.at[idx_v], sem_s)

    def _zero(i, carry):
        mbuf_v[pl.ds(i * L, L)] = jnp.zeros((L,), jnp.float32)
        return carry

    lax.fori_loop(0, V_PER_W // L, _zero, 0)

    idx_all_cp.wait()
    ub_all_cp.wait()
    bias_cp.wait()

    ones = jnp.ones((L,), jnp.float32)

    def _mark(i, carry):
        v = all_idx_v[pl.ds(i * L, L)]
        ubv = all_ub_v[pl.ds(i * L, L)]
        local = v - vbase
        inrange = (local >= 0) & (local < V_PER_W)
        safe = jnp.clip(local, 0, V_PER_W - 1)
        plsc.store_scatter(mbuf_v, [safe], ones, mask=inrange)
        plsc.store_scatter(bbuf_v, [safe], ubv, mask=inrange)
        return carry

    lax.fori_loop(0, N_DEAD // L, _mark, 0)

    pltpu.sync_copy(mbuf_v, mask_hbm.at[pl.ds(vbase, V_PER_W)])
    pltpu.sync_copy(bbuf_v, bnew_hbm.at[pl.ds(vbase, V_PER_W)])
    scatter_cp.wait()


def _sc_scatter(idx, zt, bias, ub):
    mesh = plsc.VectorSubcoreMesh(core_axis_name="c", subcore_axis_name="s")
    f = pl.kernel(
        _sc_scatter_body,
        out_type=(
            jax.ShapeDtypeStruct((D_LEARNT, BATCH), jnp.float32),
            jax.ShapeDtypeStruct((D_LEARNT,), jnp.float32),
            jax.ShapeDtypeStruct((D_LEARNT,), jnp.float32),
        ),
        mesh=mesh,
        scratch_types=[
            pltpu.VMEM((J_PER_W,), jnp.int32),
            pltpu.VMEM((J_PER_W, BATCH), jnp.float32),
            pltpu.VMEM((N_DEAD,), jnp.int32),
            pltpu.VMEM((N_DEAD,), jnp.float32),
            pltpu.VMEM((V_PER_W,), jnp.float32),
            pltpu.VMEM((V_PER_W,), jnp.float32),
            pltpu.SemaphoreType.DMA,
            pltpu.SemaphoreType.DMA,
            pltpu.SemaphoreType.DMA,
            pltpu.SemaphoreType.DMA,
            pltpu.SemaphoreType.DMA,
        ],
        compiler_params=pltpu.CompilerParams(needs_layout_passes=False),
    )
    return f(idx, zt, bias, ub)


def kernel(x, weight, bias, dictionary_vector_indices,
           updated_dictionary_weights, updated_bias_features):
    idx = dictionary_vector_indices.astype(jnp.int32)

    zt = pl.pallas_call(
        _z_body,
        grid=(N_DEAD // BLKZ,),
        in_specs=[
            pl.BlockSpec((BLKZ, D_IN), lambda t: (t, 0)),
            pl.BlockSpec((BATCH, D_IN), lambda t: (0, 0)),
        ],
        out_specs=pl.BlockSpec((BLKZ, BATCH), lambda t: (t, 0)),
        out_shape=jax.ShapeDtypeStruct((N_DEAD, BATCH), jnp.float32),
    )(updated_dictionary_weights, x)

    zfull, mask, bnew = _sc_scatter(idx, zt, bias, updated_bias_features)

    out = pl.pallas_call(
        _main_body,
        grid=(D_LEARNT // BLK,),
        in_specs=[
            pl.BlockSpec((BATCH, D_IN), lambda t: (0, 0)),
            pl.BlockSpec((BLK, D_IN), lambda t: (t, 0)),
            pl.BlockSpec((1, BLK), lambda t: (0, t)),
            pl.BlockSpec((1, BLK), lambda t: (0, t)),
            pl.BlockSpec((BLK, BATCH), lambda t: (t, 0)),
        ],
        out_specs=pl.BlockSpec((BATCH, BLK), lambda t: (0, t)),
        out_shape=jax.ShapeDtypeStruct((BATCH, D_LEARNT), jnp.float32),
    )(x, weight, bnew.reshape(1, -1), mask.reshape(1, -1), zfull)
    return out


# speedup vs baseline: 1.0035x; 1.0035x over previous
import jax
import jax.numpy as jnp
from jax import lax
from jax.experimental import pallas as pl
from jax.experimental.pallas import tpu as pltpu
from jax.experimental.pallas import tpu_sc as plsc

D_IN = 1024
D_LEARNT = 32768
BATCH = 128
N_DEAD = 4096

BLK = 2048
BLKZ = 2048

NC = 2
NS = 16
NW = NC * NS
J_PER_W = N_DEAD // NW
V_PER_W = D_LEARNT // NW
L = 16


def _z_body(u_ref, x_ref, o_ref):
    o_ref[...] = lax.dot_general(
        u_ref[...], x_ref[...], (((1,), (1,)), ((), ())),
        preferred_element_type=jnp.float32)


def _main_body(x_ref, w_ref, b_ref, m_ref, zf_ref, o_ref):
    acc = lax.dot_general(
        x_ref[...], w_ref[...], (((1,), (1,)), ((), ())),
        preferred_element_type=jnp.float32)
    zt = zf_ref[...].T
    pre = jnp.where(m_ref[...] > 0.5, zt, acc)
    o_ref[...] = jnp.maximum(pre + b_ref[...], 0.0)


def _sc_scatter_body(idx_hbm, zt_hbm, bias_hbm, ub_hbm,
                     zfull_hbm, mask_hbm, bnew_hbm,
                     idx_v, rows_v, all_idx_v, all_ub_v, mbuf_v, bbuf_v,
                     sem_r, sem_i, sem_u, sem_s, sem_b):
    wid = lax.axis_index("s") * NC + lax.axis_index("c")

    jbase = wid * J_PER_W
    rows_cp = pltpu.async_copy(
        zt_hbm.at[pl.ds(jbase, J_PER_W)], rows_v, sem_r)
    idx_all_cp = pltpu.async_copy(idx_hbm, all_idx_v, sem_i)
    ub_all_cp = pltpu.async_copy(ub_hbm, all_ub_v, sem_u)
    pltpu.sync_copy(idx_hbm.at[pl.ds(jbase, J_PER_W)], idx_v)

    vbase = wid * V_PER_W
    bias_cp = pltpu.async_copy(
        bias_hbm.at[pl.ds(vbase, V_PER_W)], bbuf_v, sem_b)

    rows_cp.wait()
    scatter_cp = pltpu.async_copy(rows_v, zfull_hbm.at[idx_v], sem_s)

    def _zero(i, carry):
        mbuf_v[pl.ds(i * L, L)] = jnp.zeros((L,), jnp.float32)
        return carry

    lax.fori_loop(0, V_PER_W // L, _zero, 0)

    idx_all_cp.wait()
    ub_all_cp.wait()
    bias_cp.wait()

    ones = jnp.ones((L,), jnp.float32)

    def _mark(i, carry):
        v = all_idx_v[pl.ds(i * L, L)]
        ubv = all_ub_v[pl.ds(i * L, L)]
        local = v - vbase
        inrange = (local >= 0) & (local < V_PER_W)
        safe = jnp.clip(local, 0, V_PER_W - 1)
        plsc.store_scatter(mbuf_v, [safe], ones, mask=inrange)
        plsc.store_scatter(bbuf_v, [safe], ubv, mask=inrange)
        return carry

    lax.fori_loop(0, N_DEAD // L, _mark, 0)

    pltpu.sync_copy(mbuf_v, mask_hbm.at[pl.ds(vbase, V_PER_W)])
    pltpu.sync_copy(bbuf_v, bnew_hbm.at[pl.ds(vbase, V_PER_W)])
    scatter_cp.wait()


def _sc_scatter(idx, zt, bias, ub):
    mesh = plsc.VectorSubcoreMesh(core_axis_name="c", subcore_axis_name="s")
    f = pl.kernel(
        _sc_scatter_body,
        out_type=(
            jax.ShapeDtypeStruct((D_LEARNT, BATCH), jnp.float32),
            jax.ShapeDtypeStruct((D_LEARNT,), jnp.float32),
            jax.ShapeDtypeStruct((D_LEARNT,), jnp.float32),
        ),
        mesh=mesh,
        scratch_types=[
            pltpu.VMEM((J_PER_W,), jnp.int32),
            pltpu.VMEM((J_PER_W, BATCH), jnp.float32),
            pltpu.VMEM((N_DEAD,), jnp.int32),
            pltpu.VMEM((N_DEAD,), jnp.float32),
            pltpu.VMEM((V_PER_W,), jnp.float32),
            pltpu.VMEM((V_PER_W,), jnp.float32),
            pltpu.SemaphoreType.DMA,
            pltpu.SemaphoreType.DMA,
            pltpu.SemaphoreType.DMA,
            pltpu.SemaphoreType.DMA,
            pltpu.SemaphoreType.DMA,
        ],
        compiler_params=pltpu.CompilerParams(needs_layout_passes=False),
    )
    return f(idx, zt, bias, ub)


def kernel(x, weight, bias, dictionary_vector_indices,
           updated_dictionary_weights, updated_bias_features):
    idx = dictionary_vector_indices.astype(jnp.int32)

    zt = pl.pallas_call(
        _z_body,
        grid=(N_DEAD // BLKZ,),
        in_specs=[
            pl.BlockSpec((BLKZ, D_IN), lambda t: (t, 0)),
            pl.BlockSpec((BATCH, D_IN), lambda t: (0, 0)),
        ],
        out_specs=pl.BlockSpec((BLKZ, BATCH), lambda t: (t, 0)),
        out_shape=jax.ShapeDtypeStruct((N_DEAD, BATCH), jnp.float32),
    )(updated_dictionary_weights, x)

    zfull, mask, bnew = _sc_scatter(idx, zt, bias, updated_bias_features)

    out = pl.pallas_call(
        _main_body,
        grid=(D_LEARNT // BLK,),
        in_specs=[
            pl.BlockSpec((BATCH, D_IN), lambda t: (0, 0)),
            pl.BlockSpec((BLK, D_IN), lambda t: (t, 0)),
            pl.BlockSpec((1, BLK), lambda t: (0, t)),
            pl.BlockSpec((1, BLK), lambda t: (0, t)),
            pl.BlockSpec((BLK, BATCH), lambda t: (t, 0)),
        ],
        out_specs=pl.BlockSpec((BATCH, BLK), lambda t: (0, t)),
        out_shape=jax.ShapeDtypeStruct((BATCH, D_LEARNT), jnp.float32),
    )(x, weight, bnew.reshape(1, -1), mask.reshape(1, -1), zfull)
    return out


# speedup vs baseline: 1.0240x; 1.0204x over previous
import jax
import jax.numpy as jnp
from jax import lax
from jax.experimental import pallas as pl
from jax.experimental.pallas import tpu as pltpu
from jax.experimental.pallas import tpu_sc as plsc

D_IN = 1024
D_LEARNT = 32768
BATCH = 128
N_DEAD = 4096

BLK = 2048
BLKZ = 2048

NC = 2
NS = 16
NW = NC * NS
J_PER_W = N_DEAD // NW
V_PER_W = D_LEARNT // NW
L = 16


def _z_body(u_ref, x_ref, o_ref):
    o_ref[...] = lax.dot_general(
        u_ref[...], x_ref[...], (((1,), (1,)), ((), ())),
        preferred_element_type=jnp.float32)


def _main_body(x_ref, w_ref, b_ref, m_ref, zf_ref, o_ref):
    acc = lax.dot_general(
        x_ref[...], w_ref[...], (((1,), (1,)), ((), ())),
        preferred_element_type=jnp.float32)
    zt = zf_ref[...].T
    pre = jnp.where(m_ref[...] > 0.5, zt, acc)
    o_ref[...] = jnp.maximum(pre + b_ref[...], 0.0)


def _sc_scatter_body(idx_hbm, zt_hbm, bias_hbm, ub_hbm,
                     zfull_hbm, mask_hbm, bnew_hbm,
                     idx_v, rows_v, all_idx_v, all_ub_v, mbuf_v, bbuf_v,
                     sem_r, sem_i, sem_u, sem_s, sem_b):
    wid = lax.axis_index("s") * NC + lax.axis_index("c")

    jbase = wid * J_PER_W
    rows_cp = pltpu.async_copy(
        zt_hbm.at[pl.ds(jbase, J_PER_W)], rows_v, sem_r)
    idx_all_cp = pltpu.async_copy(idx_hbm, all_idx_v, sem_i)
    ub_all_cp = pltpu.async_copy(ub_hbm, all_ub_v, sem_u)
    pltpu.sync_copy(idx_hbm.at[pl.ds(jbase, J_PER_W)], idx_v)

    vbase = wid * V_PER_W
    bias_cp = pltpu.async_copy(
        bias_hbm.at[pl.ds(vbase, V_PER_W)], bbuf_v, sem_b)

    rows_cp.wait()
    scatter_cp = pltpu.async_copy(rows_v, zfull_hbm.at[idx_v], sem_s)

    @plsc.parallel_loop(0, V_PER_W // L, unroll=4)
    def _zero(i):
        mbuf_v[pl.ds(i * L, L)] = jnp.zeros((L,), jnp.float32)

    idx_all_cp.wait()
    ub_all_cp.wait()
    bias_cp.wait()

    ones = jnp.ones((L,), jnp.float32)

    @plsc.parallel_loop(0, N_DEAD // L, unroll=4)
    def _mark(i):
        v = all_idx_v[pl.ds(i * L, L)]
        ubv = all_ub_v[pl.ds(i * L, L)]
        local = v - vbase
        inrange = (local >= 0) & (local < V_PER_W)
        safe = jnp.clip(local, 0, V_PER_W - 1)
        plsc.store_scatter(mbuf_v, [safe], ones, mask=inrange)
        plsc.store_scatter(bbuf_v, [safe], ubv, mask=inrange)

    pltpu.sync_copy(mbuf_v, mask_hbm.at[pl.ds(vbase, V_PER_W)])
    pltpu.sync_copy(bbuf_v, bnew_hbm.at[pl.ds(vbase, V_PER_W)])
    scatter_cp.wait()


def _sc_scatter(idx, zt, bias, ub):
    mesh = plsc.VectorSubcoreMesh(core_axis_name="c", subcore_axis_name="s")
    f = pl.kernel(
        _sc_scatter_body,
        out_type=(
            jax.ShapeDtypeStruct((D_LEARNT, BATCH), jnp.float32),
            jax.ShapeDtypeStruct((D_LEARNT,), jnp.float32),
            jax.ShapeDtypeStruct((D_LEARNT,), jnp.float32),
        ),
        mesh=mesh,
        scratch_types=[
            pltpu.VMEM((J_PER_W,), jnp.int32),
            pltpu.VMEM((J_PER_W, BATCH), jnp.float32),
            pltpu.VMEM((N_DEAD,), jnp.int32),
            pltpu.VMEM((N_DEAD,), jnp.float32),
            pltpu.VMEM((V_PER_W,), jnp.float32),
            pltpu.VMEM((V_PER_W,), jnp.float32),
            pltpu.SemaphoreType.DMA,
            pltpu.SemaphoreType.DMA,
            pltpu.SemaphoreType.DMA,
            pltpu.SemaphoreType.DMA,
            pltpu.SemaphoreType.DMA,
        ],
        compiler_params=pltpu.CompilerParams(needs_layout_passes=False),
    )
    return f(idx, zt, bias, ub)


def kernel(x, weight, bias, dictionary_vector_indices,
           updated_dictionary_weights, updated_bias_features):
    idx = dictionary_vector_indices.astype(jnp.int32)

    zt = pl.pallas_call(
        _z_body,
        grid=(N_DEAD // BLKZ,),
        in_specs=[
            pl.BlockSpec((BLKZ, D_IN), lambda t: (t, 0)),
            pl.BlockSpec((BATCH, D_IN), lambda t: (0, 0)),
        ],
        out_specs=pl.BlockSpec((BLKZ, BATCH), lambda t: (t, 0)),
        out_shape=jax.ShapeDtypeStruct((N_DEAD, BATCH), jnp.float32),
    )(updated_dictionary_weights, x)

    zfull, mask, bnew = _sc_scatter(idx, zt, bias, updated_bias_features)

    out = pl.pallas_call(
        _main_body,
        grid=(D_LEARNT // BLK,),
        in_specs=[
            pl.BlockSpec((BATCH, D_IN), lambda t: (0, 0)),
            pl.BlockSpec((BLK, D_IN), lambda t: (t, 0)),
            pl.BlockSpec((1, BLK), lambda t: (0, t)),
            pl.BlockSpec((1, BLK), lambda t: (0, t)),
            pl.BlockSpec((BLK, BATCH), lambda t: (t, 0)),
        ],
        out_specs=pl.BlockSpec((BATCH, BLK), lambda t: (0, t)),
        out_shape=jax.ShapeDtypeStruct((BATCH, D_LEARNT), jnp.float32),
    )(x, weight, bnew.reshape(1, -1), mask.reshape(1, -1), zfull)
    return out


# speedup vs baseline: 1.0248x; 1.0008x over previous
import jax
import jax.numpy as jnp
from jax import lax
from jax.experimental import pallas as pl
from jax.experimental.pallas import tpu as pltpu
from jax.experimental.pallas import tpu_sc as plsc

D_IN = 1024
D_LEARNT = 32768
BATCH = 128
N_DEAD = 4096

BLK = 2048
BLKZ = 2048

NC = 2
NS = 16
NW = NC * NS
J_PER_W = N_DEAD // NW
V_PER_W = D_LEARNT // NW
L = 16


def _z_body(u_ref, x_ref, o_ref):
    o_ref[...] = lax.dot_general(
        u_ref[...], x_ref[...], (((1,), (1,)), ((), ())),
        preferred_element_type=jnp.float32)


def _main_body(x_ref, w_ref, b_ref, m_ref, zf_ref, o_ref):
    acc = lax.dot_general(
        x_ref[...], w_ref[...], (((1,), (1,)), ((), ())),
        preferred_element_type=jnp.float32)
    zt = zf_ref[...].T
    pre = jnp.where(m_ref[...] > 0.5, zt, acc)
    o_ref[...] = jnp.maximum(pre + b_ref[...], 0.0)


def _sc_scatter_body(idx_hbm, zt_hbm, bias_hbm, ub_hbm,
                     zfull_hbm, mask_hbm, bnew_hbm,
                     idx_v, rows_v, all_idx_v, all_ub_v, mbuf_v, bbuf_v,
                     sem_r, sem_i, sem_u, sem_s, sem_b):
    wid = lax.axis_index("s") * NC + lax.axis_index("c")

    jbase = wid * J_PER_W
    rows_cp = pltpu.async_copy(
        zt_hbm.at[pl.ds(jbase, J_PER_W)], rows_v, sem_r)
    idx_all_cp = pltpu.async_copy(idx_hbm, all_idx_v, sem_i)
    ub_all_cp = pltpu.async_copy(ub_hbm, all_ub_v, sem_u)
    pltpu.sync_copy(idx_hbm.at[pl.ds(jbase, J_PER_W)], idx_v)

    vbase = wid * V_PER_W
    bias_cp = pltpu.async_copy(
        bias_hbm.at[pl.ds(vbase, V_PER_W)], bbuf_v, sem_b)

    rows_cp.wait()
    scatter_cp = pltpu.async_copy(rows_v, zfull_hbm.at[idx_v], sem_s)

    @plsc.parallel_loop(0, V_PER_W // L, unroll=4)
    def _zero(i):
        mbuf_v[pl.ds(i * L, L)] = jnp.zeros((L,), jnp.float32)

    idx_all_cp.wait()
    ub_all_cp.wait()
    bias_cp.wait()

    ones = jnp.ones((L,), jnp.float32)

    @plsc.parallel_loop(0, N_DEAD // L, unroll=8)
    def _mark(i):
        v = all_idx_v[pl.ds(i * L, L)]
        ubv = all_ub_v[pl.ds(i * L, L)]
        local = v - vbase
        inrange = (local >= 0) & (local < V_PER_W)
        safe = jnp.clip(local, 0, V_PER_W - 1)
        plsc.store_scatter(mbuf_v, [safe], ones, mask=inrange)
        plsc.store_scatter(bbuf_v, [safe], ubv, mask=inrange)

    pltpu.sync_copy(mbuf_v, mask_hbm.at[pl.ds(vbase, V_PER_W)])
    pltpu.sync_copy(bbuf_v, bnew_hbm.at[pl.ds(vbase, V_PER_W)])
    scatter_cp.wait()


def _sc_scatter(idx, zt, bias, ub):
    mesh = plsc.VectorSubcoreMesh(core_axis_name="c", subcore_axis_name="s")
    f = pl.kernel(
        _sc_scatter_body,
        out_type=(
            jax.ShapeDtypeStruct((D_LEARNT, BATCH), jnp.float32),
            jax.ShapeDtypeStruct((D_LEARNT,), jnp.float32),
            jax.ShapeDtypeStruct((D_LEARNT,), jnp.float32),
        ),
        mesh=mesh,
        scratch_types=[
            pltpu.VMEM((J_PER_W,), jnp.int32),
            pltpu.VMEM((J_PER_W, BATCH), jnp.float32),
            pltpu.VMEM((N_DEAD,), jnp.int32),
            pltpu.VMEM((N_DEAD,), jnp.float32),
            pltpu.VMEM((V_PER_W,), jnp.float32),
            pltpu.VMEM((V_PER_W,), jnp.float32),
            pltpu.SemaphoreType.DMA,
            pltpu.SemaphoreType.DMA,
            pltpu.SemaphoreType.DMA,
            pltpu.SemaphoreType.DMA,
            pltpu.SemaphoreType.DMA,
        ],
        compiler_params=pltpu.CompilerParams(needs_layout_passes=False),
    )
    return f(idx, zt, bias, ub)


def kernel(x, weight, bias, dictionary_vector_indices,
           updated_dictionary_weights, updated_bias_features):
    idx = dictionary_vector_indices.astype(jnp.int32)

    zt = pl.pallas_call(
        _z_body,
        grid=(N_DEAD // BLKZ,),
        in_specs=[
            pl.BlockSpec((BLKZ, D_IN), lambda t: (t, 0)),
            pl.BlockSpec((BATCH, D_IN), lambda t: (0, 0)),
        ],
        out_specs=pl.BlockSpec((BLKZ, BATCH), lambda t: (t, 0)),
        out_shape=jax.ShapeDtypeStruct((N_DEAD, BATCH), jnp.float32),
    )(updated_dictionary_weights, x)

    zfull, mask, bnew = _sc_scatter(idx, zt, bias, updated_bias_features)

    out = pl.pallas_call(
        _main_body,
        grid=(D_LEARNT // BLK,),
        in_specs=[
            pl.BlockSpec((BATCH, D_IN), lambda t: (0, 0)),
            pl.BlockSpec((BLK, D_IN), lambda t: (t, 0)),
            pl.BlockSpec((1, BLK), lambda t: (0, t)),
            pl.BlockSpec((1, BLK), lambda t: (0, t)),
            pl.BlockSpec((BLK, BATCH), lambda t: (t, 0)),
        ],
        out_specs=pl.BlockSpec((BATCH, BLK), lambda t: (0, t)),
        out_shape=jax.ShapeDtypeStruct((BATCH, D_LEARNT), jnp.float32),
    )(x, weight, bnew.reshape(1, -1), mask.reshape(1, -1), zfull)
    return out
